# Initial kernel scaffold; baseline (speedup 1.0000x reference)
import jax
import jax.numpy as jnp
from jax.experimental import pallas as pl


def kernel(X, edge_index, edge_weight, H, Wz, bz, Lzw, Lzb, Wr, br, Lrw, Lrb, Wh, bh, Lhw, Lhb):
    raise NotImplementedError("write your pallas kernel here")


# speedup vs baseline: 32.4586x; 32.4586x over previous
import functools

import jax
import jax.numpy as jnp
from jax import lax
from jax.experimental import pallas as pl
from jax.experimental.pallas import tpu as pltpu
from jax.experimental.pallas import tpu_sc as plsc

N_NODES = 10000
N_PAD = 10240
E_EDGES = 320000
E_PAD = 327680
EROWS = E_PAD // 128
D = 128

NC = 2
NS = 16
B = 128
NB_TILE = E_PAD // (NC * NS * B)
B_CH = 16
A_ROWS = E_PAD // (NS * 128)
A_CH = 32
DSL = N_PAD // NS


def _sc_body(src_hbm, dst_hbm, w_hbm, x_hbm, s_out, deg_out,
             s_sh, deg_sh, dst_a, w_a, src_b, dst_b, w_b,
             dinv_v, degsl, rows, scale, sem):
    cid = lax.axis_index("c")
    sid = lax.axis_index("s")
    wid = cid * NS + sid

    def _zrow(e, _):
        for c8 in range(8):
            rows[e, pl.ds(c8 * 16, 16)] = jnp.zeros((16,), jnp.float32)
        return 0
    lax.fori_loop(0, B, _zrow, 0)
    for g in range(8):
        scale[pl.ds(g * 16, 16)] = jnp.zeros((16,), jnp.float32)
    zbase = sid * DSL
    for j in range(DSL // 128):
        pltpu.sync_copy(rows, s_sh.at[pl.ds(zbase + j * 128, 128)])
        pltpu.sync_copy(scale, deg_sh.at[pl.ds(zbase + j * 128, 128)])
    plsc.subcore_barrier()

    for ch in range(A_ROWS // A_CH):
        cbase = sid * A_ROWS + ch * A_CH
        pltpu.sync_copy(dst_hbm.at[pl.ds(cbase, A_CH)], dst_a)
        pltpu.sync_copy(w_hbm.at[pl.ds(cbase, A_CH)], w_a)

        def _pha(j, _):
            pltpu.sync_copy(w_a.at[j], deg_sh.at[dst_a.at[j]], add=True)
            return 0
        lax.fori_loop(0, A_CH, _pha, 0)
    plsc.subcore_barrier()

    dbase = sid * DSL
    pltpu.sync_copy(deg_sh.at[pl.ds(dbase, DSL)], degsl)

    @pl.when(cid == 0)
    def _():
        pltpu.sync_copy(degsl, deg_out.at[pl.ds(dbase, DSL)])

    def _newt(g, _):
        sl = pl.ds(g * 16, 16)
        x = degsl[sl] + 1.0
        y = jnp.full((16,), 1.0, jnp.float32)
        for k in range(1, 11):
            y = jnp.where(x >= float(2.0 ** (2 * k - 1)),
                          jnp.float32(2.0 ** (-k)), y)
        for _ in range(5):
            y = y * (1.5 - 0.5 * x * y * y)
        degsl[sl] = y
        return 0
    lax.fori_loop(0, DSL // 16, _newt, 0)
    pltpu.sync_copy(degsl, deg_sh.at[pl.ds(dbase, DSL)])
    plsc.subcore_barrier()
    pltpu.sync_copy(deg_sh, dinv_v)

    ebase = wid * NB_TILE
    for ch in range(NB_TILE // B_CH):
        cb = ebase + ch * B_CH
        pltpu.sync_copy(src_hbm.at[pl.ds(cb, B_CH)], src_b)
        pltpu.sync_copy(dst_hbm.at[pl.ds(cb, B_CH)], dst_b)
        pltpu.sync_copy(w_hbm.at[pl.ds(cb, B_CH)], w_b)

        def _batch(b, _):
            pltpu.async_copy(x_hbm.at[src_b.at[b]], rows, sem).wait()
            for g in range(8):
                sl = pl.ds(g * 16, 16)
                dv = plsc.load_gather(dinv_v, [src_b[b, sl]])
                scale[sl] = dv * w_b[b, sl]

            def _edge(e, _):
                sv = plsc.load_gather(scale, [jnp.zeros((16,), jnp.int32) + e])
                for c8 in range(8):
                    cs = pl.ds(c8 * 16, 16)
                    rows[e, cs] = rows[e, cs] * sv
                return 0
            lax.fori_loop(0, B, _edge, 0)
            pltpu.sync_copy(rows, s_sh.at[dst_b.at[b]], add=True)
            return 0
        lax.fori_loop(0, B_CH, _batch, 0)
    plsc.subcore_barrier()

    sbase = sid * DSL
    for j in range(DSL // 128):
        sl = pl.ds(sbase + j * 128, 128)
        pltpu.sync_copy(s_sh.at[sl], s_out.at[cid, sl])


_sc_pass = functools.partial(
    pl.kernel,
    out_type=[
        jax.ShapeDtypeStruct((NC, N_PAD, D), jnp.float32),
        jax.ShapeDtypeStruct((N_PAD,), jnp.float32),
    ],
    mesh=plsc.VectorSubcoreMesh(
        core_axis_name="c", subcore_axis_name="s",
        num_cores=NC, num_subcores=NS),
    compiler_params=pltpu.CompilerParams(needs_layout_passes=False),
    scratch_types=[
        pltpu.VMEM_SHARED((N_PAD, D), jnp.float32),
        pltpu.VMEM_SHARED((N_PAD,), jnp.float32),
        pltpu.VMEM((A_CH, 128), jnp.int32),
        pltpu.VMEM((A_CH, 128), jnp.float32),
        pltpu.VMEM((B_CH, 128), jnp.int32),
        pltpu.VMEM((B_CH, 128), jnp.int32),
        pltpu.VMEM((B_CH, 128), jnp.float32),
        pltpu.VMEM((N_PAD,), jnp.float32),
        pltpu.VMEM((DSL,), jnp.float32),
        pltpu.VMEM((B, D), jnp.float32),
        pltpu.VMEM((B,), jnp.float32),
        pltpu.SemaphoreType.DMA,
    ],
)(_sc_body)


def _tc_body(s_ref, deg_ref, x_ref, h_ref,
             wz, bz, lzw, lzb, wr, br, lrw, lrb, wh, bh, lhw, lhb,
             out_ref):
    S = s_ref[0, :N_NODES, :] + s_ref[1, :N_NODES, :]
    d = lax.rsqrt(deg_ref[:N_NODES, :] + 1.0)
    X = x_ref[...]
    XA = d * S + (d * d) * X
    H = h_ref[...]

    def gate(wm, bm, lw, lb, right):
        c = jnp.dot(XA, wm[...], preferred_element_type=jnp.float32) + bm[...]
        lwv = lw[...]
        return (jnp.dot(c, lwv[:D, :], preferred_element_type=jnp.float32)
                + jnp.dot(right, lwv[D:, :], preferred_element_type=jnp.float32)
                + lb[...])

    Z = jax.nn.sigmoid(gate(wz, bz, lzw, lzb, H))
    R = jax.nn.sigmoid(gate(wr, br, lrw, lrb, H))
    Ht = jnp.tanh(gate(wh, bh, lhw, lhb, H * R))
    out_ref[...] = Z * H + (1.0 - Z) * Ht


_tc_pass = pl.pallas_call(
    _tc_body,
    out_shape=jax.ShapeDtypeStruct((N_NODES, D), jnp.float32),
)


@jax.jit
def kernel(X, edge_index, edge_weight, H,
           Wz, bz, Lzw, Lzb, Wr, br, Lrw, Lrb, Wh, bh, Lhw, Lhb):
    src = edge_index[0].astype(jnp.int32)
    dst = edge_index[1].astype(jnp.int32)
    w = edge_weight.astype(jnp.float32)
    npad = E_PAD - E_EDGES
    pad_idx = (jnp.arange(npad, dtype=jnp.int32) * 97) % N_NODES
    src_p = jnp.concatenate([src, pad_idx]).reshape(EROWS, 128)
    dst_p = jnp.concatenate([dst, pad_idx]).reshape(EROWS, 128)
    w_p = jnp.concatenate([w, jnp.zeros((npad,), jnp.float32)]).reshape(EROWS, 128)

    S, deg = _sc_pass(src_p, dst_p, w_p, X)

    return _tc_pass(S, deg.reshape(N_PAD, 1), X, H,
                    Wz, bz.reshape(1, D), Lzw, Lzb.reshape(1, D),
                    Wr, br.reshape(1, D), Lrw, Lrb.reshape(1, D),
                    Wh, bh.reshape(1, D), Lhw, Lhb.reshape(1, D))


# speedup vs baseline: 40.4047x; 1.2448x over previous
import functools

import jax
import jax.numpy as jnp
from jax import lax
from jax.experimental import pallas as pl
from jax.experimental.pallas import tpu as pltpu
from jax.experimental.pallas import tpu_sc as plsc

N_NODES = 10000
N_PAD = 10240
E_EDGES = 320000
E_PAD = 327680
B = 64
EROWS = E_PAD // B
D = 128

NC = 2
NS = 16
NB_TILE = E_PAD // (NC * NS * B)
B_CH = 16
A_ROWS = E_PAD // (NS * B)
A_CH = 16
DSL = N_PAD // NS


def _sc_body(src_hbm, dst_hbm, w_hbm, x_hbm, s_out, deg_out,
             s_sh, deg_sh, dst_a, w_a, src_b, dst_b, w_b,
             dinv_v, degsl, rows_a, rows_b, scale,
             gs_a, gs_b, ss_a, ss_b, sem_pa):
    cid = lax.axis_index("c")
    sid = lax.axis_index("s")
    wid = cid * NS + sid

    def _zrow(e, _):
        for c8 in range(8):
            rows_a[e, pl.ds(c8 * 16, 16)] = jnp.zeros((16,), jnp.float32)
        return 0
    lax.fori_loop(0, B, _zrow, 0)
    for g in range(4):
        scale[pl.ds(g * 16, 16)] = jnp.zeros((16,), jnp.float32)
    zbase = sid * DSL
    for j in range(DSL // B):
        pltpu.sync_copy(rows_a, s_sh.at[pl.ds(zbase + j * B, B)])
        pltpu.sync_copy(scale, deg_sh.at[pl.ds(zbase + j * B, B)])
    plsc.subcore_barrier()

    for ch in range(A_ROWS // A_CH):
        cbase = sid * A_ROWS + ch * A_CH
        pltpu.sync_copy(dst_hbm.at[pl.ds(cbase, A_CH)], dst_a)
        pltpu.sync_copy(w_hbm.at[pl.ds(cbase, A_CH)], w_a)

        def _grp(j, _):
            descs = []
            for k in range(8):
                b = j * 8 + k
                descs.append(pltpu.async_copy(
                    w_a.at[b], deg_sh.at[dst_a.at[b]], sem_pa, add=True))
            for dsc in descs:
                dsc.wait()
            return 0
        lax.fori_loop(0, A_CH // 8, _grp, 0)
    plsc.subcore_barrier()

    dbase = sid * DSL
    pltpu.sync_copy(deg_sh.at[pl.ds(dbase, DSL)], degsl)

    @pl.when(cid == 0)
    def _():
        pltpu.sync_copy(degsl, deg_out.at[pl.ds(dbase, DSL)])

    def _newt(g, _):
        sl = pl.ds(g * 16, 16)
        x = degsl[sl] + 1.0
        y = jnp.full((16,), 1.0, jnp.float32)
        for k in range(1, 11):
            y = jnp.where(x >= float(2.0 ** (2 * k - 1)),
                          jnp.float32(2.0 ** (-k)), y)
        for _ in range(5):
            y = y * (1.5 - 0.5 * x * y * y)
        degsl[sl] = y
        return 0
    lax.fori_loop(0, DSL // 16, _newt, 0)
    pltpu.sync_copy(degsl, deg_sh.at[pl.ds(dbase, DSL)])
    plsc.subcore_barrier()
    pltpu.sync_copy(deg_sh, dinv_v)

    def _scale_rows(rows_ref, bi):
        for g in range(4):
            sl = pl.ds(g * 16, 16)
            dv = plsc.load_gather(dinv_v, [src_b[bi, sl]])
            scale[sl] = dv * w_b[bi, sl]

        def _edge(e2, _):
            e0 = e2 * 2
            e1 = e0 + 1
            sv0 = plsc.load_gather(scale, [jnp.zeros((16,), jnp.int32) + e0])
            sv1 = plsc.load_gather(scale, [jnp.zeros((16,), jnp.int32) + e1])
            for c8 in range(8):
                cs = pl.ds(c8 * 16, 16)
                rows_ref[e0, cs] = rows_ref[e0, cs] * sv0
            for c8 in range(8):
                cs = pl.ds(c8 * 16, 16)
                rows_ref[e1, cs] = rows_ref[e1, cs] * sv1
            return 0
        lax.fori_loop(0, B // 2, _edge, 0)

    ebase = wid * NB_TILE
    for ch in range(NB_TILE // B_CH):
        cb = ebase + ch * B_CH
        pltpu.sync_copy(src_hbm.at[pl.ds(cb, B_CH)], src_b)
        pltpu.sync_copy(dst_hbm.at[pl.ds(cb, B_CH)], dst_b)
        pltpu.sync_copy(w_hbm.at[pl.ds(cb, B_CH)], w_b)
        pltpu.async_copy(x_hbm.at[src_b.at[0]], rows_a, gs_a)
        pltpu.async_copy(x_hbm.at[src_b.at[1]], rows_b, gs_b)

        def _pair(j, _):
            b0 = 2 * j
            b1 = b0 + 1
            pltpu.make_async_copy(x_hbm.at[src_b.at[b0]], rows_a, gs_a).wait()
            _scale_rows(rows_a, b0)
            sc_a = pltpu.async_copy(rows_a, s_sh.at[dst_b.at[b0]], ss_a,
                                    add=True)
            pltpu.make_async_copy(x_hbm.at[src_b.at[b1]], rows_b, gs_b).wait()
            _scale_rows(rows_b, b1)
            sc_b = pltpu.async_copy(rows_b, s_sh.at[dst_b.at[b1]], ss_b,
                                    add=True)
            sc_a.wait()

            @pl.when(j < B_CH // 2 - 1)
            def _():
                pltpu.async_copy(x_hbm.at[src_b.at[b0 + 2]], rows_a, gs_a)
            sc_b.wait()

            @pl.when(j < B_CH // 2 - 1)
            def _():
                pltpu.async_copy(x_hbm.at[src_b.at[b1 + 2]], rows_b, gs_b)
            return 0
        lax.fori_loop(0, B_CH // 2, _pair, 0)
    plsc.subcore_barrier()

    sbase = sid * DSL
    for j in range(DSL // 128):
        sl = pl.ds(sbase + j * 128, 128)
        pltpu.sync_copy(s_sh.at[sl], s_out.at[cid, sl])


_sc_pass = functools.partial(
    pl.kernel,
    out_type=[
        jax.ShapeDtypeStruct((NC, N_PAD, D), jnp.float32),
        jax.ShapeDtypeStruct((N_PAD,), jnp.float32),
    ],
    mesh=plsc.VectorSubcoreMesh(
        core_axis_name="c", subcore_axis_name="s",
        num_cores=NC, num_subcores=NS),
    compiler_params=pltpu.CompilerParams(needs_layout_passes=False),
    scratch_types=[
        pltpu.VMEM_SHARED((N_PAD, D), jnp.float32),
        pltpu.VMEM_SHARED((N_PAD,), jnp.float32),
        pltpu.VMEM((A_CH, B), jnp.int32),
        pltpu.VMEM((A_CH, B), jnp.float32),
        pltpu.VMEM((B_CH, B), jnp.int32),
        pltpu.VMEM((B_CH, B), jnp.int32),
        pltpu.VMEM((B_CH, B), jnp.float32),
        pltpu.VMEM((N_PAD,), jnp.float32),
        pltpu.VMEM((DSL,), jnp.float32),
        pltpu.VMEM((B, D), jnp.float32),
        pltpu.VMEM((B, D), jnp.float32),
        pltpu.VMEM((B,), jnp.float32),
        pltpu.SemaphoreType.DMA,
        pltpu.SemaphoreType.DMA,
        pltpu.SemaphoreType.DMA,
        pltpu.SemaphoreType.DMA,
        pltpu.SemaphoreType.DMA,
    ],
)(_sc_body)


def _tc_body(s_ref, deg_ref, x_ref, h_ref,
             wz, bz, lzw, lzb, wr, br, lrw, lrb, wh, bh, lhw, lhb,
             out_ref):
    S = s_ref[0, :N_NODES, :] + s_ref[1, :N_NODES, :]
    d = lax.rsqrt(deg_ref[:N_NODES, :] + 1.0)
    X = x_ref[...]
    XA = d * S + (d * d) * X
    H = h_ref[...]

    def gate(wm, bm, lw, lb, right):
        c = jnp.dot(XA, wm[...], preferred_element_type=jnp.float32) + bm[...]
        lwv = lw[...]
        return (jnp.dot(c, lwv[:D, :], preferred_element_type=jnp.float32)
                + jnp.dot(right, lwv[D:, :], preferred_element_type=jnp.float32)
                + lb[...])

    Z = jax.nn.sigmoid(gate(wz, bz, lzw, lzb, H))
    R = jax.nn.sigmoid(gate(wr, br, lrw, lrb, H))
    Ht = jnp.tanh(gate(wh, bh, lhw, lhb, H * R))
    out_ref[...] = Z * H + (1.0 - Z) * Ht


_tc_pass = pl.pallas_call(
    _tc_body,
    out_shape=jax.ShapeDtypeStruct((N_NODES, D), jnp.float32),
)


@jax.jit
def kernel(X, edge_index, edge_weight, H,
           Wz, bz, Lzw, Lzb, Wr, br, Lrw, Lrb, Wh, bh, Lhw, Lhb):
    src = edge_index[0].astype(jnp.int32)
    dst = edge_index[1].astype(jnp.int32)
    w = edge_weight.astype(jnp.float32)
    npad = E_PAD - E_EDGES
    pad_idx = (jnp.arange(npad, dtype=jnp.int32) * 97) % N_NODES
    src_p = jnp.concatenate([src, pad_idx]).reshape(EROWS, B)
    dst_p = jnp.concatenate([dst, pad_idx]).reshape(EROWS, B)
    w_p = jnp.concatenate([w, jnp.zeros((npad,), jnp.float32)]).reshape(EROWS, B)

    S, deg = _sc_pass(src_p, dst_p, w_p, X)

    return _tc_pass(S, deg.reshape(N_PAD, 1), X, H,
                    Wz, bz.reshape(1, D), Lzw, Lzb.reshape(1, D),
                    Wr, br.reshape(1, D), Lrw, Lrb.reshape(1, D),
                    Wh, bh.reshape(1, D), Lhw, Lhb.reshape(1, D))


# speedup vs baseline: 46.2124x; 1.1437x over previous
import functools

import jax
import jax.numpy as jnp
from jax import lax
from jax.experimental import pallas as pl
from jax.experimental.pallas import tpu as pltpu
from jax.experimental.pallas import tpu_sc as plsc

N_NODES = 10000
N_PAD = 10240
E_EDGES = 320000
E_PAD = 327680
B = 64
EROWS = E_PAD // B
D = 128

NC = 2
NS = 16
NB_TILE = E_PAD // (NC * NS * B)
B_CH = 32
A_ROWS = E_PAD // (NS * 128)
A_CH = 16
DSL = N_PAD // NS


def _sc_body(src_hbm, dst_pa, w_pa, dst_s, w_hbm, x_hbm, s_out, deg_out,
             s_sh, deg_sh, dst_a, w_a, src_b, dst_v, w_b,
             dinv_v, degsl, rows_a, rows_b, scale,
             gs_a, gs_b, ss_a, ss_b, sem_pa):
    cid = lax.axis_index("c")
    sid = lax.axis_index("s")
    wid = cid * NS + sid

    def _zrow(e, _):
        for c8 in range(8):
            rows_a[e, pl.ds(c8 * 16, 16)] = jnp.zeros((16,), jnp.float32)
        return 0
    lax.fori_loop(0, B, _zrow, 0)
    for g in range(4):
        scale[pl.ds(g * 16, 16)] = jnp.zeros((16,), jnp.float32)
    zbase = sid * DSL
    for j in range(DSL // B):
        pltpu.sync_copy(rows_a, s_sh.at[pl.ds(zbase + j * B, B)])
        pltpu.sync_copy(scale, deg_sh.at[pl.ds(zbase + j * B, B)])
    plsc.subcore_barrier()

    def _acnk(ch, _):
        cbase = sid * A_ROWS + ch * A_CH
        pltpu.sync_copy(dst_pa.at[pl.ds(cbase, A_CH)], dst_a)
        pltpu.sync_copy(w_pa.at[pl.ds(cbase, A_CH)], w_a)

        def _grp(j, _):
            descs = []
            for k in range(8):
                b = j * 8 + k
                descs.append(pltpu.async_copy(
                    w_a.at[b], deg_sh.at[dst_a.at[b]], sem_pa, add=True))
            for dsc in descs:
                dsc.wait()
            return 0
        lax.fori_loop(0, A_CH // 8, _grp, 0)
        return 0
    lax.fori_loop(0, A_ROWS // A_CH, _acnk, 0)
    plsc.subcore_barrier()

    dbase = sid * DSL
    pltpu.sync_copy(deg_sh.at[pl.ds(dbase, DSL)], degsl)

    @pl.when(cid == 0)
    def _():
        pltpu.sync_copy(degsl, deg_out.at[pl.ds(dbase, DSL)])

    def _newt(g, _):
        sl = pl.ds(g * 16, 16)
        x = degsl[sl] + 1.0
        y = jnp.full((16,), 1.0, jnp.float32)
        for k in range(1, 11):
            y = jnp.where(x >= float(2.0 ** (2 * k - 1)),
                          jnp.float32(2.0 ** (-k)), y)
        for _ in range(5):
            y = y * (1.5 - 0.5 * x * y * y)
        degsl[sl] = y
        return 0
    lax.fori_loop(0, DSL // 16, _newt, 0)
    pltpu.sync_copy(degsl, deg_sh.at[pl.ds(dbase, DSL)])
    plsc.subcore_barrier()
    pltpu.sync_copy(deg_sh, dinv_v)

    def _scale_half(rows_ref, bi, h):
        for g in range(2):
            sl = pl.ds(h * 32 + g * 16, 16)
            dv = plsc.load_gather(dinv_v, [src_b[bi, sl]])
            scale[sl] = dv * w_b[bi, sl]

        def _edge(e2):
            e0 = h * 32 + e2 * 2
            e1 = e0 + 1
            sv0 = plsc.load_gather(scale, [jnp.zeros((16,), jnp.int32) + e0])
            sv1 = plsc.load_gather(scale, [jnp.zeros((16,), jnp.int32) + e1])
            for c8 in range(8):
                cs = pl.ds(c8 * 16, 16)
                rows_ref[e0, cs] = rows_ref[e0, cs] * sv0
            for c8 in range(8):
                cs = pl.ds(c8 * 16, 16)
                rows_ref[e1, cs] = rows_ref[e1, cs] * sv1
            return None
        plsc.parallel_loop(0, 16, unroll=2)(_edge)

    ebase = wid * NB_TILE

    def _bcnk(ch, _):
        cb = ebase + ch * B_CH
        pltpu.sync_copy(src_hbm.at[pl.ds(cb, B_CH)], src_b)
        pltpu.sync_copy(dst_s.at[pl.ds(2 * cb, 2 * B_CH)], dst_v)
        pltpu.sync_copy(w_hbm.at[pl.ds(cb, B_CH)], w_b)
        pltpu.async_copy(x_hbm.at[src_b.at[0]], rows_a, gs_a)
        pltpu.async_copy(x_hbm.at[src_b.at[1]], rows_b, gs_b)

        def _pair(j, _):
            b0 = 2 * j
            b1 = b0 + 1
            pltpu.make_async_copy(x_hbm.at[src_b.at[b0]], rows_a, gs_a).wait()
            _scale_half(rows_a, b0, 0)
            sa1 = pltpu.async_copy(rows_a.at[pl.ds(0, 32)],
                                   s_sh.at[dst_v.at[2 * b0]], ss_a, add=True)
            _scale_half(rows_a, b0, 1)
            sa2 = pltpu.async_copy(rows_a.at[pl.ds(32, 32)],
                                   s_sh.at[dst_v.at[2 * b0 + 1]], ss_a,
                                   add=True)
            pltpu.make_async_copy(x_hbm.at[src_b.at[b1]], rows_b, gs_b).wait()
            _scale_half(rows_b, b1, 0)
            sb1 = pltpu.async_copy(rows_b.at[pl.ds(0, 32)],
                                   s_sh.at[dst_v.at[2 * b1]], ss_b, add=True)
            _scale_half(rows_b, b1, 1)
            sb2 = pltpu.async_copy(rows_b.at[pl.ds(32, 32)],
                                   s_sh.at[dst_v.at[2 * b1 + 1]], ss_b,
                                   add=True)
            sa1.wait()
            sa2.wait()

            @pl.when(j < B_CH // 2 - 1)
            def _():
                pltpu.async_copy(x_hbm.at[src_b.at[b0 + 2]], rows_a, gs_a)
            sb1.wait()
            sb2.wait()

            @pl.when(j < B_CH // 2 - 1)
            def _():
                pltpu.async_copy(x_hbm.at[src_b.at[b1 + 2]], rows_b, gs_b)
            return 0
        lax.fori_loop(0, B_CH // 2, _pair, 0)
        return 0
    lax.fori_loop(0, NB_TILE // B_CH, _bcnk, 0)
    plsc.subcore_barrier()

    sbase = sid * DSL
    for j in range(DSL // 128):
        sl = pl.ds(sbase + j * 128, 128)
        pltpu.sync_copy(s_sh.at[sl], s_out.at[cid, sl])


_sc_pass = functools.partial(
    pl.kernel,
    out_type=[
        jax.ShapeDtypeStruct((NC, N_PAD, D), jnp.float32),
        jax.ShapeDtypeStruct((N_PAD,), jnp.float32),
    ],
    mesh=plsc.VectorSubcoreMesh(
        core_axis_name="c", subcore_axis_name="s",
        num_cores=NC, num_subcores=NS),
    compiler_params=pltpu.CompilerParams(needs_layout_passes=False),
    scratch_types=[
        pltpu.VMEM_SHARED((N_PAD, D), jnp.float32),
        pltpu.VMEM_SHARED((N_PAD,), jnp.float32),
        pltpu.VMEM((A_CH, 128), jnp.int32),
        pltpu.VMEM((A_CH, 128), jnp.float32),
        pltpu.VMEM((B_CH, B), jnp.int32),
        pltpu.VMEM((2 * B_CH, 32), jnp.int32),
        pltpu.VMEM((B_CH, B), jnp.float32),
        pltpu.VMEM((N_PAD,), jnp.float32),
        pltpu.VMEM((DSL,), jnp.float32),
        pltpu.VMEM((B, D), jnp.float32),
        pltpu.VMEM((B, D), jnp.float32),
        pltpu.VMEM((B,), jnp.float32),
        pltpu.SemaphoreType.DMA,
        pltpu.SemaphoreType.DMA,
        pltpu.SemaphoreType.DMA,
        pltpu.SemaphoreType.DMA,
        pltpu.SemaphoreType.DMA,
    ],
)(_sc_body)


def _tc_body(s_ref, deg_ref, x_ref, h_ref,
             wz, bz, lzw, lzb, wr, br, lrw, lrb, wh, bh, lhw, lhb,
             out_ref):
    S = s_ref[0, :N_NODES, :] + s_ref[1, :N_NODES, :]
    d = lax.rsqrt(deg_ref[:N_NODES, :] + 1.0)
    X = x_ref[...]
    XA = d * S + (d * d) * X
    H = h_ref[...]

    def gate(wm, bm, lw, lb, right):
        c = jnp.dot(XA, wm[...], preferred_element_type=jnp.float32) + bm[...]
        lwv = lw[...]
        return (jnp.dot(c, lwv[:D, :], preferred_element_type=jnp.float32)
                + jnp.dot(right, lwv[D:, :], preferred_element_type=jnp.float32)
                + lb[...])

    Z = jax.nn.sigmoid(gate(wz, bz, lzw, lzb, H))
    R = jax.nn.sigmoid(gate(wr, br, lrw, lrb, H))
    Ht = jnp.tanh(gate(wh, bh, lhw, lhb, H * R))
    out_ref[...] = Z * H + (1.0 - Z) * Ht


_tc_pass = pl.pallas_call(
    _tc_body,
    out_shape=jax.ShapeDtypeStruct((N_NODES, D), jnp.float32),
)


@jax.jit
def kernel(X, edge_index, edge_weight, H,
           Wz, bz, Lzw, Lzb, Wr, br, Lrw, Lrb, Wh, bh, Lhw, Lhb):
    src = edge_index[0].astype(jnp.int32)
    dst = edge_index[1].astype(jnp.int32)
    w = edge_weight.astype(jnp.float32)
    npad = E_PAD - E_EDGES
    pad_idx = (jnp.arange(npad, dtype=jnp.int32) * 97) % N_NODES
    src_p = jnp.concatenate([src, pad_idx]).reshape(EROWS, B)
    dst_full = jnp.concatenate([dst, pad_idx])
    w_full = jnp.concatenate([w, jnp.zeros((npad,), jnp.float32)])

    S, deg = _sc_pass(src_p,
                      dst_full.reshape(E_PAD // 128, 128),
                      w_full.reshape(E_PAD // 128, 128),
                      dst_full.reshape(E_PAD // 32, 32),
                      w_full.reshape(EROWS, B),
                      X)

    return _tc_pass(S, deg.reshape(N_PAD, 1), X, H,
                    Wz, bz.reshape(1, D), Lzw, Lzb.reshape(1, D),
                    Wr, br.reshape(1, D), Lrw, Lrb.reshape(1, D),
                    Wh, bh.reshape(1, D), Lhw, Lhb.reshape(1, D))
